# baseline (speedup 1.0000x reference)
import functools

import jax
import jax.numpy as jnp
from jax import lax
from jax.experimental import pallas as pl
from jax.experimental.pallas import tpu as pltpu
from jax.experimental.pallas import tpu_sc as plsc

N_NODES = 10000
N_EDGES = 320000
D_FEAT = 128
LEAKY = 0.5

NC = 2
NS = 16
NW = NC * NS
EW = N_EDGES // NW
CH = 80
NG = EW // CH
GNB = 3
FNB = 3
NBR = 8
RBLK = 624
ZBLK = 16

_DNUMS = lax.GatherDimensionNumbers(
    offset_dims=(), collapsed_slice_dims=(0,), start_index_map=(0,))


def _gate_body(e_ref, z_ref, x_ref):
    z = z_ref[...]
    v = e_ref[...] * (2.0 * jax.nn.sigmoid(z) - 1.0)
    r = v.shape[0]
    p = jnp.transpose(v.reshape(r, 4, 2, 16), (0, 1, 3, 2)).reshape(r, D_FEAT)
    x_ref[...] = p.astype(jnp.bfloat16)


def _combine_body(p_ref, o_ref):
    s = p_ref[0] + p_ref[1]
    o_ref[...] = jnp.where(s >= 0.0, s, LEAKY * s)


def _spmm_body(rows_hbm, cols_hbm, vals_hbm, x_hbm, out_hbm,
               cols_b, vals_b, rows_b, mbf_v, mf_v, zeros_v, agg_sh,
               semc, semv, semr, semg, sems):
    c = lax.axis_index("c")
    s = lax.axis_index("s")
    wid = s * NC + c
    ebase = wid * EW

    zvec = jnp.zeros((16,), jnp.float32)

    def _zero_body(i, _):
        r = i // 8
        cc = (i % 8) * 16
        zeros_v[r, pl.ds(cc, 16)] = zvec
        return 0

    lax.fori_loop(0, ZBLK * 8, _zero_body, 0)

    rbase = s * RBLK

    def _zero_dma(k, _):
        pltpu.sync_copy(zeros_v, agg_sh.at[pl.ds(rbase + k * ZBLK, ZBLK)])
        return 0

    lax.fori_loop(0, RBLK // ZBLK, _zero_dma, 0)

    @pl.when(s == NS - 1)
    def _zero_tail():
        pltpu.sync_copy(zeros_v, agg_sh.at[pl.ds(NS * RBLK, N_NODES - NS * RBLK)])

    plsc.subcore_barrier()

    def _issue_idx(g):
        off = ebase + g * CH
        pltpu.async_copy(cols_hbm.at[pl.ds(off, CH)], cols_b.at[g % GNB], semc)
        pltpu.async_copy(vals_hbm.at[pl.ds(off, CH)], vals_b.at[g % GNB], semv)
        pltpu.async_copy(rows_hbm.at[pl.ds(off, CH)], rows_b.at[g % NBR], semr)

    def _wait_cols(b):
        pltpu.make_async_copy(cols_hbm.at[pl.ds(ebase, CH)], cols_b.at[b],
                              semc).wait()

    def _issue_gather(b):
        pltpu.async_copy(x_hbm.at[cols_b.at[b]], mbf_v.at[b], semg)

    def _wait_gather(b):
        pltpu.make_async_copy(x_hbm.at[cols_b.at[b]], mbf_v.at[b], semg).wait()

    def _issue_scatter(b, rb):
        pltpu.async_copy(mf_v.at[b], agg_sh.at[rows_b.at[rb]], sems, add=True)

    def _wait_scatter(b, rb):
        pltpu.make_async_copy(mf_v.at[b], agg_sh.at[rows_b.at[rb]],
                              sems).wait()

    for g in range(3):
        _issue_idx(g)
    _wait_cols(0)
    _issue_gather(0)
    _wait_cols(1)
    _issue_gather(1)

    def _chunk_body(g, _):
        bg = g % GNB
        bf = g % FNB
        _wait_gather(bg)
        pltpu.make_async_copy(vals_hbm.at[pl.ds(ebase, CH)], vals_b.at[bg],
                              semv).wait()

        @pl.when(g >= FNB)
        def _free_f32():
            _wait_scatter(bf, (g - FNB) % NBR)

        src = mbf_v.at[bg]
        dst = mf_v.at[bf]
        for eg in range(CH // 16):
            v16 = vals_b[bg, pl.ds(eg * 16, 16)]
            for e16 in range(16):
                bc = lax.gather(
                    v16, jnp.full((16, 1), e16, jnp.int32), _DNUMS, (1,),
                    mode=lax.GatherScatterMode.PROMISE_IN_BOUNDS)
                r = eg * 16 + e16
                for q in range(4):
                    w = src[r, pl.ds(q * 16, 16)]
                    lo = lax.bitcast_convert_type(w << 16, jnp.float32)
                    hi = lax.bitcast_convert_type(w & jnp.int32(-65536),
                                                  jnp.float32)
                    dst[r, pl.ds(q * 32, 16)] = lo * bc
                    dst[r, pl.ds(q * 32 + 16, 16)] = hi * bc

        pltpu.make_async_copy(rows_hbm.at[pl.ds(ebase, CH)],
                              rows_b.at[g % NBR], semr).wait()
        _issue_scatter(bf, g % NBR)

        @pl.when(g + 2 < NG)
        def _pref():
            _wait_cols((g + 2) % GNB)
            _issue_gather((g + 2) % GNB)

        @pl.when(g + 3 < NG)
        def _idx():
            _issue_idx(g + 3)

        return 0

    lax.fori_loop(0, NG, _chunk_body, 0)

    for g in range(NG - FNB, NG):
        _wait_scatter(g % FNB, g % NBR)

    plsc.subcore_barrier()

    pltpu.sync_copy(agg_sh.at[pl.ds(rbase, RBLK)],
                    out_hbm.at[c].at[pl.ds(rbase, RBLK)])

    @pl.when(s == NS - 1)
    def _write_tail():
        pltpu.sync_copy(agg_sh.at[pl.ds(NS * RBLK, N_NODES - NS * RBLK)],
                        out_hbm.at[c].at[pl.ds(NS * RBLK, N_NODES - NS * RBLK)])


@functools.partial(
    pl.kernel,
    mesh=plsc.VectorSubcoreMesh(core_axis_name="c", subcore_axis_name="s"),
    compiler_params=pltpu.CompilerParams(use_tc_tiling_on_sc=False),
    out_type=jax.ShapeDtypeStruct((NC, N_NODES, D_FEAT), jnp.float32),
    scratch_types=[
        pltpu.VMEM((GNB, CH), jnp.int32),
        pltpu.VMEM((GNB, CH), jnp.float32),
        pltpu.VMEM((NBR, CH), jnp.int32),
        pltpu.VMEM((GNB, CH, D_FEAT // 2), jnp.int32),
        pltpu.VMEM((FNB, CH, D_FEAT), jnp.float32),
        pltpu.VMEM((ZBLK, D_FEAT), jnp.float32),
        pltpu.VMEM_SHARED((N_NODES, D_FEAT), jnp.float32),
        pltpu.SemaphoreType.DMA,
        pltpu.SemaphoreType.DMA,
        pltpu.SemaphoreType.DMA,
        pltpu.SemaphoreType.DMA,
        pltpu.SemaphoreType.DMA,
    ],
)
def _spmm_sc(rows_hbm, cols_hbm, vals_hbm, x_hbm, out_hbm, *scratch):
    _spmm_body(rows_hbm, cols_hbm, vals_hbm, x_hbm, out_hbm, *scratch)


def kernel(adj_edge_index, adj_values, embeds, zishiying):
    rows = adj_edge_index[0]
    cols = adj_edge_index[1]

    x = pl.pallas_call(
        _gate_body,
        grid=(5,),
        in_specs=[pl.BlockSpec((2000, D_FEAT), lambda i: (i, 0)),
                  pl.BlockSpec((2000, D_FEAT), lambda i: (i, 0))],
        out_specs=pl.BlockSpec((2000, D_FEAT), lambda i: (i, 0)),
        out_shape=jax.ShapeDtypeStruct((N_NODES, D_FEAT), jnp.bfloat16),
    )(embeds, zishiying)

    x_packed = lax.bitcast_convert_type(
        x.reshape(N_NODES, D_FEAT // 2, 2), jnp.int32)

    partials = _spmm_sc(rows, cols, adj_values, x_packed)

    out = pl.pallas_call(
        _combine_body,
        grid=(5,),
        in_specs=[pl.BlockSpec((NC, 2000, D_FEAT), lambda i: (0, i, 0))],
        out_specs=pl.BlockSpec((2000, D_FEAT), lambda i: (i, 0)),
        out_shape=jax.ShapeDtypeStruct((N_NODES, D_FEAT), jnp.float32),
    )(partials)
    return out


# speedup vs baseline: 3.8220x; 3.8220x over previous
import functools

import jax
import jax.numpy as jnp
from jax import lax
from jax.experimental import pallas as pl
from jax.experimental.pallas import tpu as pltpu
from jax.experimental.pallas import tpu_sc as plsc

N_NODES = 10000
N_EDGES = 320000
D_FEAT = 128
LEAKY = 0.5

NC = 2
NS = 16
NW = NC * NS
EW = N_EDGES // NW
CH = 80
NG = EW // CH
NB = 4
NBR = 8
RBLK = 624
ZBLK = 48

_DNUMS = lax.GatherDimensionNumbers(
    offset_dims=(), collapsed_slice_dims=(0,), start_index_map=(0,))


def _gate_body(e_ref, z_ref, x_ref):
    z = z_ref[...]
    x_ref[...] = e_ref[...] * (2.0 * jax.nn.sigmoid(z) - 1.0)


def _combine_body(p_ref, o_ref):
    s = p_ref[0] + p_ref[1]
    o_ref[...] = jnp.where(s >= 0.0, s, LEAKY * s)


def _spmm_body(rows_hbm, cols_hbm, vals_hbm, x_hbm, out_hbm,
               cols_b, vals_b, rows_b, msgs_v, zeros_v, agg_sh,
               semc, semv, semr, semg, sems, semz):
    c = lax.axis_index("c")
    s = lax.axis_index("s")
    wid = s * NC + c
    ebase = wid * EW

    zvec = jnp.zeros((16,), jnp.float32)

    def _zero_body(i, _):
        r = i // 8
        cc = (i % 8) * 16
        zeros_v[r, pl.ds(cc, 16)] = zvec
        return 0

    lax.fori_loop(0, ZBLK * 8, _zero_body, 0)

    rbase = s * RBLK

    def _zero_dma(k, _):
        pltpu.async_copy(zeros_v, agg_sh.at[pl.ds(rbase + k * ZBLK, ZBLK)],
                         semz)
        return 0

    lax.fori_loop(0, RBLK // ZBLK, _zero_dma, 0)

    @pl.when(s == NS - 1)
    def _zero_tail():
        pltpu.async_copy(zeros_v.at[pl.ds(0, N_NODES - NS * RBLK)],
                         agg_sh.at[pl.ds(NS * RBLK, N_NODES - NS * RBLK)],
                         semz)

    def _zero_drain(k, _):
        pltpu.make_async_copy(zeros_v,
                              agg_sh.at[pl.ds(rbase, ZBLK)], semz).wait()
        return 0

    lax.fori_loop(0, RBLK // ZBLK, _zero_drain, 0)

    @pl.when(s == NS - 1)
    def _zero_tail_drain():
        pltpu.make_async_copy(zeros_v.at[pl.ds(0, N_NODES - NS * RBLK)],
                              agg_sh.at[pl.ds(NS * RBLK, N_NODES - NS * RBLK)],
                              semz).wait()

    plsc.subcore_barrier()

    def _issue_idx(g):
        off = ebase + g * CH
        pltpu.async_copy(cols_hbm.at[pl.ds(off, CH)], cols_b.at[g % NB], semc)
        pltpu.async_copy(vals_hbm.at[pl.ds(off, CH)], vals_b.at[g % NB], semv)
        pltpu.async_copy(rows_hbm.at[pl.ds(off, CH)], rows_b.at[g % NBR], semr)

    def _wait_cols(b):
        pltpu.make_async_copy(cols_hbm.at[pl.ds(ebase, CH)], cols_b.at[b],
                              semc).wait()

    def _issue_gather(b):
        pltpu.async_copy(x_hbm.at[cols_b.at[b]], msgs_v.at[b], semg)

    def _wait_gather(b):
        pltpu.make_async_copy(x_hbm.at[cols_b.at[b]], msgs_v.at[b], semg).wait()

    def _issue_scatter(b, rb):
        pltpu.async_copy(msgs_v.at[b], agg_sh.at[rows_b.at[rb]], sems,
                         add=True)

    def _wait_scatter(b, rb):
        pltpu.make_async_copy(msgs_v.at[b], agg_sh.at[rows_b.at[rb]],
                              sems).wait()

    for g in range(3):
        _issue_idx(g)
    _wait_cols(0)
    _issue_gather(0)
    _wait_cols(1)
    _issue_gather(1)

    def _chunk_body(g, _):
        b = g % NB
        b2 = (g + 2) % NB
        _wait_gather(b)
        pltpu.make_async_copy(vals_hbm.at[pl.ds(ebase, CH)], vals_b.at[b],
                              semv).wait()

        mb = msgs_v.at[b]
        for eg in range(CH // 16):
            v16 = vals_b[b, pl.ds(eg * 16, 16)]
            for e16 in range(16):
                bc = lax.gather(
                    v16, jnp.full((16, 1), e16, jnp.int32), _DNUMS, (1,),
                    mode=lax.GatherScatterMode.PROMISE_IN_BOUNDS)
                r = eg * 16 + e16
                for j in range(8):
                    mb[r, pl.ds(j * 16, 16)] = mb[r, pl.ds(j * 16, 16)] * bc

        pltpu.make_async_copy(rows_hbm.at[pl.ds(ebase, CH)],
                              rows_b.at[g % NBR], semr).wait()
        _issue_scatter(b, g % NBR)

        @pl.when(g + 2 < NG)
        def _pref():
            @pl.when(g >= 2)
            def _free():
                _wait_scatter(b2, (g - 2) % NBR)
            _wait_cols(b2)
            _issue_gather(b2)

        @pl.when(g + 3 < NG)
        def _idx():
            _issue_idx(g + 3)

        return 0

    lax.fori_loop(0, NG, _chunk_body, 0)

    for g in range(NG - NB, NG):
        _wait_scatter(g % NB, g % NBR)

    plsc.subcore_barrier()

    pltpu.sync_copy(agg_sh.at[pl.ds(rbase, RBLK)],
                    out_hbm.at[c].at[pl.ds(rbase, RBLK)])

    @pl.when(s == NS - 1)
    def _write_tail():
        pltpu.sync_copy(agg_sh.at[pl.ds(NS * RBLK, N_NODES - NS * RBLK)],
                        out_hbm.at[c].at[pl.ds(NS * RBLK, N_NODES - NS * RBLK)])


@functools.partial(
    pl.kernel,
    mesh=plsc.VectorSubcoreMesh(core_axis_name="c", subcore_axis_name="s"),
    out_type=jax.ShapeDtypeStruct((NC, N_NODES, D_FEAT), jnp.float32),
    scratch_types=[
        pltpu.VMEM((NB, CH), jnp.int32),
        pltpu.VMEM((NB, CH), jnp.float32),
        pltpu.VMEM((NBR, CH), jnp.int32),
        pltpu.VMEM((NB, CH, D_FEAT), jnp.float32),
        pltpu.VMEM((ZBLK, D_FEAT), jnp.float32),
        pltpu.VMEM_SHARED((N_NODES, D_FEAT), jnp.float32),
        pltpu.SemaphoreType.DMA,
        pltpu.SemaphoreType.DMA,
        pltpu.SemaphoreType.DMA,
        pltpu.SemaphoreType.DMA,
        pltpu.SemaphoreType.DMA,
        pltpu.SemaphoreType.DMA,
    ],
)
def _spmm_sc(rows_hbm, cols_hbm, vals_hbm, x_hbm, out_hbm, *scratch):
    _spmm_body(rows_hbm, cols_hbm, vals_hbm, x_hbm, out_hbm, *scratch)


def kernel(adj_edge_index, adj_values, embeds, zishiying):
    rows = adj_edge_index[0]
    cols = adj_edge_index[1]

    x = pl.pallas_call(
        _gate_body,
        grid=(5,),
        in_specs=[pl.BlockSpec((2000, D_FEAT), lambda i: (i, 0)),
                  pl.BlockSpec((2000, D_FEAT), lambda i: (i, 0))],
        out_specs=pl.BlockSpec((2000, D_FEAT), lambda i: (i, 0)),
        out_shape=jax.ShapeDtypeStruct((N_NODES, D_FEAT), jnp.float32),
    )(embeds, zishiying)

    partials = _spmm_sc(rows, cols, adj_values, x)

    out = pl.pallas_call(
        _combine_body,
        grid=(5,),
        in_specs=[pl.BlockSpec((NC, 2000, D_FEAT), lambda i: (0, i, 0))],
        out_specs=pl.BlockSpec((2000, D_FEAT), lambda i: (i, 0)),
        out_shape=jax.ShapeDtypeStruct((N_NODES, D_FEAT), jnp.float32),
    )(partials)
    return out


# speedup vs baseline: 4.0463x; 1.0587x over previous
import functools

import jax
import jax.numpy as jnp
from jax import lax
from jax.experimental import pallas as pl
from jax.experimental.pallas import tpu as pltpu
from jax.experimental.pallas import tpu_sc as plsc

N_NODES = 10000
N_EDGES = 320000
D_FEAT = 128
LEAKY = 0.5

NC = 2
NS = 16
NW = NC * NS
EW = N_EDGES // NW
CH = 40
NG = EW // CH
NB = 8
NBR = 16
GAH = 4
IAH = 6
RBLK = 624
ZBLK = 24

_DNUMS = lax.GatherDimensionNumbers(
    offset_dims=(), collapsed_slice_dims=(0,), start_index_map=(0,))


def _gate_body(e_ref, z_ref, x_ref):
    z = z_ref[...]
    x_ref[...] = e_ref[...] * (2.0 * jax.nn.sigmoid(z) - 1.0)


def _combine_body(p_ref, o_ref):
    s = p_ref[0] + p_ref[1]
    o_ref[...] = jnp.where(s >= 0.0, s, LEAKY * s)


def _spmm_body(rows_hbm, cols_hbm, vals_hbm, x_hbm, out_hbm,
               cols_b, vals_b, rows_b, msgs_v, zeros_v, agg_sh,
               semc, semv, semr, semg, sems, semz):
    c = lax.axis_index("c")
    s = lax.axis_index("s")
    wid = s * NC + c
    ebase = wid * EW

    zvec = jnp.zeros((16,), jnp.float32)

    def _zero_body(i, _):
        r = i // 8
        cc = (i % 8) * 16
        zeros_v[r, pl.ds(cc, 16)] = zvec
        return 0

    lax.fori_loop(0, ZBLK * 8, _zero_body, 0)

    rbase = s * RBLK

    def _zero_dma(k, _):
        pltpu.async_copy(zeros_v, agg_sh.at[pl.ds(rbase + k * ZBLK, ZBLK)],
                         semz)
        return 0

    lax.fori_loop(0, RBLK // ZBLK, _zero_dma, 0)

    @pl.when(s == NS - 1)
    def _zero_tail():
        pltpu.async_copy(zeros_v.at[pl.ds(0, N_NODES - NS * RBLK)],
                         agg_sh.at[pl.ds(NS * RBLK, N_NODES - NS * RBLK)],
                         semz)

    def _zero_drain(k, _):
        pltpu.make_async_copy(zeros_v,
                              agg_sh.at[pl.ds(rbase, ZBLK)], semz).wait()
        return 0

    lax.fori_loop(0, RBLK // ZBLK, _zero_drain, 0)

    @pl.when(s == NS - 1)
    def _zero_tail_drain():
        pltpu.make_async_copy(zeros_v.at[pl.ds(0, N_NODES - NS * RBLK)],
                              agg_sh.at[pl.ds(NS * RBLK, N_NODES - NS * RBLK)],
                              semz).wait()

    plsc.subcore_barrier()

    def _issue_idx(g):
        off = ebase + g * CH
        pltpu.async_copy(cols_hbm.at[pl.ds(off, CH)], cols_b.at[g % NB], semc)
        pltpu.async_copy(vals_hbm.at[pl.ds(off, CH)], vals_b.at[g % NB], semv)
        pltpu.async_copy(rows_hbm.at[pl.ds(off, CH)], rows_b.at[g % NBR], semr)

    def _wait_cols(b):
        pltpu.make_async_copy(cols_hbm.at[pl.ds(ebase, CH)], cols_b.at[b],
                              semc).wait()

    def _issue_gather(b):
        pltpu.async_copy(x_hbm.at[cols_b.at[b]], msgs_v.at[b], semg)

    def _wait_gather(b):
        pltpu.make_async_copy(x_hbm.at[cols_b.at[b]], msgs_v.at[b], semg).wait()

    def _issue_scatter(b, rb):
        pltpu.async_copy(msgs_v.at[b], agg_sh.at[rows_b.at[rb]], sems,
                         add=True)

    def _wait_scatter(b, rb):
        pltpu.make_async_copy(msgs_v.at[b], agg_sh.at[rows_b.at[rb]],
                              sems).wait()

    for g in range(IAH):
        _issue_idx(g)
    for g in range(GAH):
        _wait_cols(g)
        _issue_gather(g)

    def _chunk_body(g, _):
        b = g % NB
        bn = (g + GAH) % NB
        _wait_gather(b)
        pltpu.make_async_copy(vals_hbm.at[pl.ds(ebase, CH)], vals_b.at[b],
                              semv).wait()

        mb = msgs_v.at[b]
        for base, lanes in ((0, range(16)), (16, range(16)), (24, range(8, 16))):
            v16 = vals_b[b, pl.ds(base, 16)]
            for e16 in lanes:
                bc = lax.gather(
                    v16, jnp.full((16, 1), e16, jnp.int32), _DNUMS, (1,),
                    mode=lax.GatherScatterMode.PROMISE_IN_BOUNDS)
                r = base + e16
                for j in range(8):
                    mb[r, pl.ds(j * 16, 16)] = mb[r, pl.ds(j * 16, 16)] * bc

        pltpu.make_async_copy(rows_hbm.at[pl.ds(ebase, CH)],
                              rows_b.at[g % NBR], semr).wait()
        _issue_scatter(b, g % NBR)

        @pl.when(g + GAH < NG)
        def _pref():
            @pl.when(g >= GAH)
            def _free():
                _wait_scatter(bn, (g - GAH) % NBR)
            _wait_cols(bn)
            _issue_gather(bn)

        @pl.when(g + IAH < NG)
        def _idx():
            _issue_idx(g + IAH)

        return 0

    lax.fori_loop(0, NG, _chunk_body, 0)

    for g in range(NG - 2 * GAH, NG):
        _wait_scatter(g % NB, g % NBR)

    plsc.subcore_barrier()

    pltpu.sync_copy(agg_sh.at[pl.ds(rbase, RBLK)],
                    out_hbm.at[c].at[pl.ds(rbase, RBLK)])

    @pl.when(s == NS - 1)
    def _write_tail():
        pltpu.sync_copy(agg_sh.at[pl.ds(NS * RBLK, N_NODES - NS * RBLK)],
                        out_hbm.at[c].at[pl.ds(NS * RBLK, N_NODES - NS * RBLK)])


@functools.partial(
    pl.kernel,
    mesh=plsc.VectorSubcoreMesh(core_axis_name="c", subcore_axis_name="s"),
    out_type=jax.ShapeDtypeStruct((NC, N_NODES, D_FEAT), jnp.float32),
    scratch_types=[
        pltpu.VMEM((NB, CH), jnp.int32),
        pltpu.VMEM((NB, CH), jnp.float32),
        pltpu.VMEM((NBR, CH), jnp.int32),
        pltpu.VMEM((NB, CH, D_FEAT), jnp.float32),
        pltpu.VMEM((ZBLK, D_FEAT), jnp.float32),
        pltpu.VMEM_SHARED((N_NODES, D_FEAT), jnp.float32),
        pltpu.SemaphoreType.DMA,
        pltpu.SemaphoreType.DMA,
        pltpu.SemaphoreType.DMA,
        pltpu.SemaphoreType.DMA,
        pltpu.SemaphoreType.DMA,
        pltpu.SemaphoreType.DMA,
    ],
)
def _spmm_sc(rows_hbm, cols_hbm, vals_hbm, x_hbm, out_hbm, *scratch):
    _spmm_body(rows_hbm, cols_hbm, vals_hbm, x_hbm, out_hbm, *scratch)


def kernel(adj_edge_index, adj_values, embeds, zishiying):
    rows = adj_edge_index[0]
    cols = adj_edge_index[1]

    x = pl.pallas_call(
        _gate_body,
        grid=(5,),
        in_specs=[pl.BlockSpec((2000, D_FEAT), lambda i: (i, 0)),
                  pl.BlockSpec((2000, D_FEAT), lambda i: (i, 0))],
        out_specs=pl.BlockSpec((2000, D_FEAT), lambda i: (i, 0)),
        out_shape=jax.ShapeDtypeStruct((N_NODES, D_FEAT), jnp.float32),
    )(embeds, zishiying)

    partials = _spmm_sc(rows, cols, adj_values, x)

    out = pl.pallas_call(
        _combine_body,
        grid=(5,),
        in_specs=[pl.BlockSpec((NC, 2000, D_FEAT), lambda i: (0, i, 0))],
        out_specs=pl.BlockSpec((2000, D_FEAT), lambda i: (i, 0)),
        out_shape=jax.ShapeDtypeStruct((N_NODES, D_FEAT), jnp.float32),
    )(partials)
    return out


# speedup vs baseline: 4.0750x; 1.0071x over previous
import functools

import jax
import jax.numpy as jnp
from jax import lax
from jax.experimental import pallas as pl
from jax.experimental.pallas import tpu as pltpu
from jax.experimental.pallas import tpu_sc as plsc

N_NODES = 10000
N_EDGES = 320000
D_FEAT = 128
LEAKY = 0.5

NC = 2
NS = 16
NW = NC * NS
EW = N_EDGES // NW
CH = 40
NG = EW // CH
NB = 8
NBR = 16
GAH = 4
IAH = 6
RBLK = 624
ZBLK = 24

_DNUMS = lax.GatherDimensionNumbers(
    offset_dims=(), collapsed_slice_dims=(0,), start_index_map=(0,))


def _gate_body(e_ref, z_ref, x_ref):
    z = z_ref[...]
    x_ref[...] = e_ref[...] * (2.0 * jax.nn.sigmoid(z) - 1.0)


def _combine_body(p_ref, o_ref):
    s = p_ref[0] + p_ref[1]
    o_ref[...] = jnp.where(s >= 0.0, s, LEAKY * s)


def _spmm_body(rows_hbm, cols_hbm, vals_hbm, x_hbm, out_hbm,
               cols_b, vals_b, rows_b, msgs_v, zeros_v, agg_sh,
               semc, semv, semr, semg, sems, semz):
    c = lax.axis_index("c")
    s = lax.axis_index("s")
    wid = s * NC + c
    ebase = wid * EW

    zvec = jnp.zeros((16,), jnp.float32)

    def _zero_body(i, _):
        r = i // 8
        cc = (i % 8) * 16
        zeros_v[r, pl.ds(cc, 16)] = zvec
        return 0

    lax.fori_loop(0, ZBLK * 8, _zero_body, 0)

    rbase = s * RBLK

    def _zero_dma(k, _):
        pltpu.async_copy(zeros_v, agg_sh.at[pl.ds(rbase + k * ZBLK, ZBLK)],
                         semz)
        return 0

    lax.fori_loop(0, RBLK // ZBLK, _zero_dma, 0)

    @pl.when(s == NS - 1)
    def _zero_tail():
        pltpu.async_copy(zeros_v.at[pl.ds(0, N_NODES - NS * RBLK)],
                         agg_sh.at[pl.ds(NS * RBLK, N_NODES - NS * RBLK)],
                         semz)

    def _zero_drain(k, _):
        pltpu.make_async_copy(zeros_v,
                              agg_sh.at[pl.ds(rbase, ZBLK)], semz).wait()
        return 0

    lax.fori_loop(0, RBLK // ZBLK, _zero_drain, 0)

    @pl.when(s == NS - 1)
    def _zero_tail_drain():
        pltpu.make_async_copy(zeros_v.at[pl.ds(0, N_NODES - NS * RBLK)],
                              agg_sh.at[pl.ds(NS * RBLK, N_NODES - NS * RBLK)],
                              semz).wait()

    plsc.subcore_barrier()

    def _issue_idx(g):
        off = ebase + g * CH
        pltpu.async_copy(cols_hbm.at[pl.ds(off, CH)], cols_b.at[g % NB], semc)
        pltpu.async_copy(vals_hbm.at[pl.ds(off, CH)], vals_b.at[g % NB], semv)
        pltpu.async_copy(rows_hbm.at[pl.ds(off, CH)], rows_b.at[g % NBR], semr)

    def _wait_cols(b):
        pltpu.make_async_copy(cols_hbm.at[pl.ds(ebase, CH)], cols_b.at[b],
                              semc).wait()

    def _issue_gather(b):
        pltpu.async_copy(x_hbm.at[cols_b.at[b]], msgs_v.at[b], semg)

    def _wait_gather(b):
        pltpu.make_async_copy(x_hbm.at[cols_b.at[b]], msgs_v.at[b], semg).wait()

    def _issue_scatter(b, rb):
        pltpu.async_copy(msgs_v.at[b], agg_sh.at[rows_b.at[rb]], sems,
                         add=True)

    def _wait_scatter(b, rb):
        pltpu.make_async_copy(msgs_v.at[b], agg_sh.at[rows_b.at[rb]],
                              sems).wait()

    for g in range(IAH):
        _issue_idx(g)
    for g in range(GAH):
        _wait_cols(g)
        _issue_gather(g)

    def _chunk_body(g, _):
        b = g % NB
        bn = (g + GAH) % NB
        _wait_gather(b)
        pltpu.make_async_copy(vals_hbm.at[pl.ds(ebase, CH)], vals_b.at[b],
                              semv).wait()

        mb = msgs_v.at[b]
        for base, lanes in ((0, range(16)), (16, range(16)), (24, range(8, 16))):
            v16 = vals_b[b, pl.ds(base, 16)]
            for e16 in lanes:
                bc = lax.gather(
                    v16, jnp.full((16, 1), e16, jnp.int32), _DNUMS, (1,),
                    mode=lax.GatherScatterMode.PROMISE_IN_BOUNDS)
                r = base + e16
                for j in range(8):
                    mb[r, pl.ds(j * 16, 16)] = mb[r, pl.ds(j * 16, 16)] * bc

        pltpu.make_async_copy(rows_hbm.at[pl.ds(ebase, CH)],
                              rows_b.at[g % NBR], semr).wait()
        _issue_scatter(b, g % NBR)

        @pl.when(g + GAH < NG)
        def _pref():
            @pl.when(g >= GAH)
            def _free():
                _wait_scatter(bn, (g - GAH) % NBR)
            _wait_cols(bn)
            _issue_gather(bn)

        @pl.when(g + IAH < NG)
        def _idx():
            _issue_idx(g + IAH)

        return 0

    lax.fori_loop(0, NG, _chunk_body, 0)

    for g in range(NG - 2 * GAH, NG):
        _wait_scatter(g % NB, g % NBR)

    plsc.subcore_barrier()

    pltpu.sync_copy(agg_sh.at[pl.ds(rbase, RBLK)],
                    out_hbm.at[c].at[pl.ds(rbase, RBLK)])

    @pl.when(s == NS - 1)
    def _write_tail():
        pltpu.sync_copy(agg_sh.at[pl.ds(NS * RBLK, N_NODES - NS * RBLK)],
                        out_hbm.at[c].at[pl.ds(NS * RBLK, N_NODES - NS * RBLK)])


@functools.partial(
    pl.kernel,
    mesh=plsc.VectorSubcoreMesh(core_axis_name="c", subcore_axis_name="s"),
    out_type=jax.ShapeDtypeStruct((NC, N_NODES, D_FEAT), jnp.float32),
    scratch_types=[
        pltpu.VMEM((NB, CH), jnp.int32),
        pltpu.VMEM((NB, CH), jnp.float32),
        pltpu.VMEM((NBR, CH), jnp.int32),
        pltpu.VMEM((NB, CH, D_FEAT), jnp.float32),
        pltpu.VMEM((ZBLK, D_FEAT), jnp.float32),
        pltpu.VMEM_SHARED((N_NODES, D_FEAT), jnp.float32),
        pltpu.SemaphoreType.DMA,
        pltpu.SemaphoreType.DMA,
        pltpu.SemaphoreType.DMA,
        pltpu.SemaphoreType.DMA,
        pltpu.SemaphoreType.DMA,
        pltpu.SemaphoreType.DMA,
    ],
)
def _spmm_sc(rows_hbm, cols_hbm, vals_hbm, x_hbm, out_hbm, *scratch):
    _spmm_body(rows_hbm, cols_hbm, vals_hbm, x_hbm, out_hbm, *scratch)


def kernel(adj_edge_index, adj_values, embeds, zishiying):
    rows = adj_edge_index[0]
    cols = adj_edge_index[1]

    x = pl.pallas_call(
        _gate_body,
        out_shape=jax.ShapeDtypeStruct((N_NODES, D_FEAT), jnp.float32),
    )(embeds, zishiying)

    partials = _spmm_sc(rows, cols, adj_values, x)

    out = pl.pallas_call(
        _combine_body,
        out_shape=jax.ShapeDtypeStruct((N_NODES, D_FEAT), jnp.float32),
    )(partials)
    return out
